# baseline (speedup 1.0000x reference)
import jax
import jax.numpy as jnp
from jax.experimental import pallas as pl


_BLOCK_S = 512


def _add_kernel(x_ref, pos_ref, out_ref):
    out_ref[...] = x_ref[...] + pos_ref[...]


def _part(x, pos, b0_blocks, nb, seq_len, d_model):
    grid = (seq_len // _BLOCK_S,)
    return pl.pallas_call(
        _add_kernel,
        grid=grid,
        in_specs=[
            pl.BlockSpec((nb, _BLOCK_S, d_model), lambda s: (b0_blocks, s, 0)),
            pl.BlockSpec((_BLOCK_S, d_model), lambda s: (s, 0)),
        ],
        out_specs=pl.BlockSpec((nb, _BLOCK_S, d_model), lambda s: (0, s, 0)),
        out_shape=jax.ShapeDtypeStruct((nb, seq_len, d_model), x.dtype),
    )(x, pos)


def kernel(x, pos_emb):
    batch, seq_len, d_model = x.shape
    pos = pos_emb[:seq_len]
    a = _part(x, pos, 0, 3, seq_len, d_model)
    b = _part(x, pos, 3, 1, seq_len, d_model)
    return jnp.concatenate([a, b], axis=0)


# speedup vs baseline: 1.2653x; 1.2653x over previous
import functools

import jax
import jax.numpy as jnp
from jax import lax
from jax.experimental import pallas as pl
from jax.experimental.pallas import tpu as pltpu
from jax.experimental.pallas import tpu_sc as plsc
from jax.experimental.compute_on import compute_on2


_BLOCK_S = 512
_C = 32
_NW = 32
_LANES = 16


def _tc_add_kernel(x_ref, pos_ref, out_ref):
    out_ref[...] = x_ref[...] + pos_ref[...]


def _tc_batches012(x, pos, seq_len, d_model):
    return pl.pallas_call(
        _tc_add_kernel,
        grid=(seq_len // _BLOCK_S, 3),
        in_specs=[
            pl.BlockSpec((1, _BLOCK_S, d_model), lambda s, b: (b, s, 0)),
            pl.BlockSpec((_BLOCK_S, d_model), lambda s, b: (s, 0)),
        ],
        out_specs=pl.BlockSpec((1, _BLOCK_S, d_model), lambda s, b: (b, s, 0)),
        out_shape=jax.ShapeDtypeStruct((4, seq_len, d_model), x.dtype),
    )(x, pos)


def _sc_batch3_body(x_hbm, pos_hbm, out_hbm, pos_v, x_v):
    wid = lax.axis_index("s") * 2 + lax.axis_index("c")
    rows_per_worker = 8192 // _NW
    s_base = wid * rows_per_worker

    def chunk_body(ci, carry):
        s0 = s_base + ci * _C
        pltpu.sync_copy(pos_hbm.at[pl.ds(s0, _C)], pos_v)
        pltpu.sync_copy(x_hbm.at[3, pl.ds(s0, _C)], x_v)

        def row_body(r, c2):
            for i in range(1024 // _LANES):
                sl = pl.ds(i * _LANES, _LANES)
                x_v[r, sl] = x_v[r, sl] + pos_v[r, sl]
            return c2

        lax.fori_loop(0, _C, row_body, 0)
        pltpu.sync_copy(x_v, out_hbm.at[0, pl.ds(s0, _C)])
        return carry

    lax.fori_loop(0, rows_per_worker // _C, chunk_body, 0)


def _sc_batch3(x, pos, seq_len, d_model):
    mesh = plsc.VectorSubcoreMesh(core_axis_name="c", subcore_axis_name="s")
    fn = functools.partial(
        pl.kernel,
        mesh=mesh,
        out_type=jax.ShapeDtypeStruct((1, seq_len, d_model), jnp.float32),
        scratch_types=[
            pltpu.VMEM((_C, d_model), jnp.float32),
            pltpu.VMEM((_C, d_model), jnp.float32),
        ],
    )(_sc_batch3_body)
    from jax._src import core as _jcore
    wrapped = compute_on2(
        lambda xx, pp: fn(xx, pp),
        compute_type="tpu_sparsecore",
        out_memory_spaces=_jcore.MemorySpace.Device,
    )
    return wrapped(x, pos)


def kernel(x, pos_emb):
    batch, seq_len, d_model = x.shape
    pos = pos_emb[:seq_len]
    tc_full = _tc_batches012(x, pos, seq_len, d_model)
    sc3 = _sc_batch3(x, pos, seq_len, d_model)
    tc_full, sc3 = lax.optimization_barrier((tc_full, sc3))
    return lax.dynamic_update_slice(tc_full, sc3, (3, 0, 0))


# speedup vs baseline: 2.0234x; 1.5991x over previous
import jax
import jax.numpy as jnp
from jax.experimental import pallas as pl


_BLOCK_S = 256


def _add_kernel(x_ref, pos_ref, out_ref):
    out_ref[...] = x_ref[...] + pos_ref[...]


def kernel(x, pos_emb):
    batch, seq_len, d_model = x.shape
    pos = pos_emb[:seq_len]
    grid = (seq_len // _BLOCK_S,)
    return pl.pallas_call(
        _add_kernel,
        grid=grid,
        in_specs=[
            pl.BlockSpec((batch, _BLOCK_S, d_model), lambda s: (0, s, 0)),
            pl.BlockSpec((_BLOCK_S, d_model), lambda s: (s, 0)),
        ],
        out_specs=pl.BlockSpec((batch, _BLOCK_S, d_model), lambda s: (0, s, 0)),
        out_shape=jax.ShapeDtypeStruct(x.shape, x.dtype),
    )(x, pos)


# speedup vs baseline: 2.0270x; 1.0018x over previous
import jax
import jax.numpy as jnp
from jax.experimental import pallas as pl


_BLOCK_S = 512


def _add_kernel(x_ref, pos_ref, out_ref):
    out_ref[...] = x_ref[...] + pos_ref[...]


def kernel(x, pos_emb):
    batch, seq_len, d_model = x.shape
    pos = pos_emb[:seq_len]
    grid = (seq_len // _BLOCK_S,)
    return pl.pallas_call(
        _add_kernel,
        grid=grid,
        in_specs=[
            pl.BlockSpec((batch, _BLOCK_S, d_model), lambda s: (0, s, 0)),
            pl.BlockSpec((_BLOCK_S, d_model), lambda s: (s, 0)),
        ],
        out_specs=pl.BlockSpec((batch, _BLOCK_S, d_model), lambda s: (0, s, 0)),
        out_shape=jax.ShapeDtypeStruct(x.shape, x.dtype),
    )(x, pos)
